# baseline (speedup 1.0000x reference)
import jax
import jax.numpy as jnp
from jax import lax
from jax.experimental import pallas as pl
from jax.experimental.pallas import tpu as pltpu

VOCAB = 1000000
EMBED_DIM = 64
BATCH = 16384
NUM_CLASSES = 128

BM = 2048
NB = BATCH // BM
NQ = 8
UNROLL = 4


def _body(idx_sref, emb_ref, w_ref, b_ref, lab_ref, out_ref, xbuf, sems):
    i = pl.program_id(0)

    def issue_block(block, slot):
        def issue_one(j8, _):
            base = j8 * NQ
            for q in range(NQ):
                r = idx_sref[block * BM + base + q]
                pltpu.make_async_copy(
                    emb_ref.at[pl.ds(r, 1), :],
                    xbuf.at[slot, pl.ds(base + q, 1), :],
                    sems.at[slot, q],
                ).start()
            return 0
        lax.fori_loop(0, BM // NQ, issue_one, 0, unroll=UNROLL)

    @pl.when(i == 0)
    def _():
        issue_block(0, 0)

    @pl.when(i + 1 < NB)
    def _():
        issue_block(i + 1, (i + 1) % 2)

    slot = i % 2
    for q in range(NQ):
        pltpu.make_async_copy(
            emb_ref.at[pl.ds(0, BM // NQ), :],
            xbuf.at[slot, pl.ds(0, BM // NQ), :],
            sems.at[slot, q],
        ).wait()

    x = xbuf[slot]
    w = w_ref[...]
    bias = b_ref[...]
    lab = lab_ref[0, 0, :]
    scores = jnp.dot(x, w, preferred_element_type=jnp.float32) + bias
    m = jnp.max(scores, axis=-1, keepdims=True)
    lse = jnp.log(jnp.sum(jnp.exp(scores - m), axis=-1, keepdims=True)) + m
    cls = lax.broadcasted_iota(jnp.int32, scores.shape, 1)
    picked = jnp.sum(
        jnp.where(cls == lab[:, None], scores, 0.0), axis=-1, keepdims=True
    )
    part = jnp.sum(lse - picked, axis=0, keepdims=True) * (1.0 / BATCH)

    @pl.when(i == 0)
    def _():
        out_ref[...] = part

    @pl.when(i > 0)
    def _():
        out_ref[...] = out_ref[...] + part


def kernel(sentence_features, labels, emb, W, b):
    idx = sentence_features.astype(jnp.int32)
    labels3 = labels.astype(jnp.int32).reshape(NB, 1, BM)
    grid_spec = pltpu.PrefetchScalarGridSpec(
        num_scalar_prefetch=1,
        grid=(NB,),
        in_specs=[
            pl.BlockSpec(memory_space=pltpu.HBM),
            pl.BlockSpec((EMBED_DIM, NUM_CLASSES), lambda i, *_: (0, 0)),
            pl.BlockSpec((1, NUM_CLASSES), lambda i, *_: (0, 0)),
            pl.BlockSpec((1, 1, BM), lambda i, *_: (i, 0, 0)),
        ],
        out_specs=pl.BlockSpec((1, 1), lambda i, *_: (0, 0)),
        scratch_shapes=[
            pltpu.VMEM((2, BM, EMBED_DIM), jnp.float32),
            pltpu.SemaphoreType.DMA((2, NQ)),
        ],
    )
    loss = pl.pallas_call(
        _body,
        grid_spec=grid_spec,
        out_shape=jax.ShapeDtypeStruct((1, 1), jnp.float32),
    )(idx, emb, W, b.reshape(1, NUM_CLASSES), labels3)
    return loss[0, 0]
